# baseline (speedup 1.0000x reference)
import functools

import jax
import jax.numpy as jnp
from jax import lax
from jax.experimental import pallas as pl
from jax.experimental.pallas import tpu as pltpu
from jax.experimental.pallas import tpu_sc as plsc


def _gather_rows_sc(h2, gidx, n_rows):
    D = h2.shape[1]
    mesh = plsc.VectorSubcoreMesh(core_axis_name="c", subcore_axis_name="s")

    @functools.partial(
        pl.kernel,
        mesh=mesh,
        out_type=jax.ShapeDtypeStruct((n_rows, D), jnp.float32),
        scratch_types=[
            pltpu.VMEM((16,), jnp.int32),
            pltpu.VMEM((16, D), jnp.float32),
            pltpu.SemaphoreType.DMA,
        ],
    )
    def k(h2_hbm, gidx_hbm, out_hbm, idx_v, rows_v, sem):
        wid = lax.axis_index("s") * 2 + lax.axis_index("c")
        pltpu.sync_copy(gidx_hbm.at[wid], idx_v)
        pltpu.async_copy(h2_hbm.at[idx_v], rows_v, sem).wait()
        pltpu.sync_copy(rows_v.at[pl.ds(0, 8)], out_hbm.at[pl.ds(wid * 8, 8)])
        pltpu.sync_copy(rows_v.at[pl.ds(8, 1)], out_hbm.at[pl.ds(256 + wid, 1)])

    return k(h2, gidx)


def _compute_tc(rows, W1, b1, W2, b2, Wr_pad, br_pad, u_row, a_col, rrf,
                B, S):
    K, D, D4 = W1.shape
    R, RM = 256, 32

    def body(rows_ref, W1_ref, b1_ref, W2_ref, b2_ref, Wr_ref, br_ref,
             urow_ref, acol_ref, rr_ref,
             val_ref, M_scr, w_scr, acc_scr, wb_scr):
        k = pl.program_id(0)

        @pl.when(k == 0)
        def _init():
            u_r = urow_ref[...]
            a_c = acol_ref[...]
            rr = rr_ref[0, 0]
            absd = jnp.abs(u_r - a_c)
            bc = lax.broadcasted_iota(jnp.int32, (RM, R), 0) // 16
            brw = lax.broadcasted_iota(jnp.int32, (RM, R), 1) // 128
            adj = (absd > 0.5) & (absd < rr + 0.5) & (bc == brw)
            adjf = adj.astype(jnp.float32)
            cnt = jnp.sum(adjf, axis=1, keepdims=True)
            M_scr[...] = adjf / jnp.maximum(cnt, 1.0)
            ha = rows_ref[R:R + RM, :]
            logits = jnp.dot(ha, Wr_ref[...],
                             preferred_element_type=jnp.float32) + br_ref[...]
            mx = jnp.max(logits, axis=1, keepdims=True)
            e = jnp.exp(logits - mx)
            w_scr[...] = e / jnp.sum(e, axis=1, keepdims=True)
            acc_scr[...] = jnp.zeros_like(acc_scr)
            wb_scr[...] = jnp.zeros_like(wb_scr)

        hts = rows_ref[0:R, :]
        h1 = jnp.dot(hts, W1_ref[0],
                     preferred_element_type=jnp.float32) + b1_ref[0]
        H = h1 * 0.5 * (1.0 + lax.erf(h1 * 0.7071067811865476))
        P = jnp.dot(M_scr[...], H, preferred_element_type=jnp.float32)
        lane = lax.broadcasted_iota(jnp.int32, (RM, 128), 1)
        wcol = jnp.sum(jnp.where(lane == k, w_scr[...], 0.0),
                       axis=1, keepdims=True)
        acc_scr[...] += jnp.dot(P * wcol, W2_ref[0],
                                preferred_element_type=jnp.float32)
        wb_scr[...] += wcol * b2_ref[0]

        @pl.when(k == K - 1)
        def _fin():
            s_col = (jnp.sum(M_scr[...], axis=1, keepdims=True) > 0.5)
            eo = acc_scr[...] + s_col.astype(jnp.float32) * wb_scr[...]
            mu = jnp.mean(eo, axis=1, keepdims=True)
            var = jnp.mean((eo - mu) ** 2, axis=1, keepdims=True)
            val_ref[...] = (eo - mu) * lax.rsqrt(var + 1e-5)

    return pl.pallas_call(
        body,
        grid=(K,),
        in_specs=[
            pl.BlockSpec((R + RM, D), lambda k: (0, 0)),
            pl.BlockSpec((1, D, D4), lambda k: (k, 0, 0)),
            pl.BlockSpec((1, 1, D4), lambda k: (k, 0, 0)),
            pl.BlockSpec((1, D4, D), lambda k: (k, 0, 0)),
            pl.BlockSpec((1, 1, D), lambda k: (k, 0, 0)),
            pl.BlockSpec((D, 128), lambda k: (0, 0)),
            pl.BlockSpec((1, 128), lambda k: (0, 0)),
            pl.BlockSpec((1, R), lambda k: (0, 0)),
            pl.BlockSpec((RM, 1), lambda k: (0, 0)),
            pl.BlockSpec((1, 1), lambda k: (0, 0)),
        ],
        out_specs=pl.BlockSpec((RM, D), lambda k: (0, 0)),
        out_shape=jax.ShapeDtypeStruct((RM, D), jnp.float32),
        scratch_shapes=[
            pltpu.VMEM((RM, R), jnp.float32),
            pltpu.VMEM((RM, 128), jnp.float32),
            pltpu.VMEM((RM, D), jnp.float32),
            pltpu.VMEM((RM, D), jnp.float32),
        ],
        compiler_params=pltpu.CompilerParams(
            dimension_semantics=("arbitrary",)),
    )(rows, W1, b1.reshape(K, 1, D4), W2, b2.reshape(K, 1, D),
      Wr_pad, br_pad, u_row, a_col, rrf)


def _rowwrite_tc(delta0, val, win32, tb32, tr32, B, S, D):
    RM = val.shape[0]

    def body(out_in_ref, val_ref, win_ref, tb_ref, tr_ref, out_ref, rsem):
        for j in range(RM):
            @pl.when(win_ref[j] != 0)
            def _row(j=j):
                pltpu.make_async_copy(
                    val_ref.at[pl.ds(j, 1)],
                    out_ref.at[tb_ref[j]].at[pl.ds(tr_ref[j], 1)],
                    rsem.at[j]).start()
        for j in range(RM):
            @pl.when(win_ref[j] != 0)
            def _roww(j=j):
                pltpu.make_async_copy(
                    val_ref.at[pl.ds(j, 1)],
                    out_ref.at[tb_ref[j]].at[pl.ds(tr_ref[j], 1)],
                    rsem.at[j]).wait()

    return pl.pallas_call(
        body,
        in_specs=[
            pl.BlockSpec(memory_space=pl.ANY),
            pl.BlockSpec((RM, D), lambda: (0, 0)),
            pl.BlockSpec(memory_space=pltpu.SMEM),
            pl.BlockSpec(memory_space=pltpu.SMEM),
            pl.BlockSpec(memory_space=pltpu.SMEM),
        ],
        out_specs=pl.BlockSpec(memory_space=pl.ANY),
        out_shape=jax.ShapeDtypeStruct((B, S, D), jnp.float32),
        input_output_aliases={0: 0},
        scratch_shapes=[pltpu.SemaphoreType.DMA((RM,))],
    )(delta0, val, win32, tb32, tr32)


def _routed_mixture(rows, mask_indices, unmasked_indices, range_r, Wr, br,
                    W1, b1, W2, b2, B, S):
    K, D, _ = W1.shape
    uf = unmasked_indices.astype(jnp.float32).reshape(-1)
    af = mask_indices.astype(jnp.float32).reshape(-1)
    u_row = uf.reshape(1, -1)
    a_col = af.reshape(-1, 1)
    rrf = jnp.asarray(range_r, jnp.float32).reshape(1, 1)
    Wr_pad = jnp.concatenate([Wr, jnp.zeros((D, 128 - K), Wr.dtype)], axis=1)
    br_pad = jnp.concatenate(
        [br, jnp.full((128 - K,), -1e9, br.dtype)]).reshape(1, 128)
    a_i = mask_indices.astype(jnp.int32)
    u_i = unmasked_indices.astype(jnp.int32)
    ad = jnp.abs(a_i[:, :, None] - u_i[:, None, :])
    s = ((ad >= 1) & (ad <= range_r)).sum(-1) > 0
    jj = jnp.arange(a_i.shape[1], dtype=jnp.int32)
    eqm = a_i[:, :, None] == a_i[:, None, :]
    exists = (eqm & (jj[None, :] > jj[:, None])[None] & s[:, None, :]).any(-1)
    win32 = (s & ~exists).reshape(-1).astype(jnp.int32)
    tb32 = jnp.repeat(jnp.arange(B, dtype=jnp.int32), a_i.shape[1])
    tr32 = a_i.reshape(-1)
    val = _compute_tc(rows, W1, b1, W2, b2, Wr_pad, br_pad, u_row, a_col,
                      rrf, B, S)
    delta0 = jnp.zeros((B, S, D), jnp.float32)
    return _rowwrite_tc(delta0, val, win32, tb32, tr32, B, S, D)


def kernel(h_L, mask_indices, unmasked_indices, range_r, Wr, br, W1, b1, W2,
           b2):
    B, S, D = h_L.shape
    offs = (jnp.arange(B, dtype=jnp.int32) * S)[:, None]
    ug = (unmasked_indices.astype(jnp.int32) + offs).reshape(32, 8)
    ag = (mask_indices.astype(jnp.int32) + offs).reshape(32, 1)
    gidx = jnp.concatenate([ug, jnp.broadcast_to(ag, (32, 8))], axis=1)
    rows = _gather_rows_sc(h_L.reshape(B * S, D), gidx, 288)
    return _routed_mixture(rows, mask_indices, unmasked_indices, range_r, Wr,
                           br, W1, b1, W2, b2, B, S)


# speedup vs baseline: 1.0598x; 1.0598x over previous
import functools

import jax
import jax.numpy as jnp
from jax import lax
from jax.experimental import pallas as pl
from jax.experimental.pallas import tpu as pltpu
from jax.experimental.pallas import tpu_sc as plsc


def _gather_rows_sc(h2, gidx, n_rows):
    D = h2.shape[1]
    mesh = plsc.VectorSubcoreMesh(core_axis_name="c", subcore_axis_name="s")

    @functools.partial(
        pl.kernel,
        mesh=mesh,
        out_type=jax.ShapeDtypeStruct((n_rows, D), jnp.float32),
        scratch_types=[
            pltpu.VMEM((16,), jnp.int32),
            pltpu.VMEM((16, D), jnp.float32),
            pltpu.SemaphoreType.DMA,
        ],
    )
    def k(h2_hbm, gidx_hbm, out_hbm, idx_v, rows_v, sem):
        wid = lax.axis_index("s") * 2 + lax.axis_index("c")
        pltpu.sync_copy(gidx_hbm.at[wid], idx_v)
        pltpu.async_copy(h2_hbm.at[idx_v], rows_v, sem).wait()
        pltpu.sync_copy(rows_v.at[pl.ds(0, 8)], out_hbm.at[pl.ds(wid * 8, 8)])
        pltpu.sync_copy(rows_v.at[pl.ds(8, 1)], out_hbm.at[pl.ds(256 + wid, 1)])

    return k(h2, gidx)


def _compute_tc(rows, W1, b1, W2, b2, Wr_pad, br_pad, u_row, a_col, rrf,
                win32, tb32, tr32, B, S):
    K, D, D4 = W1.shape
    R, RM, ZB = 256, 32, 512
    NZ = B * S // ZB

    def body(rows_ref, W1_ref, b1_ref, W2_ref, b2_ref, Wr_ref, br_ref,
             urow_ref, acol_ref, rr_ref, win_ref, tb_ref, tr_ref,
             out_ref, M_scr, w_scr, acc_scr, wb_scr, val_scr, zblk,
             zsem, rsem):
        k = pl.program_id(0)

        @pl.when(k == 0)
        def _init():
            u_r = urow_ref[...]
            a_c = acol_ref[...]
            rr = rr_ref[0, 0]
            zblk[...] = jnp.zeros_like(zblk)
            absd = jnp.abs(u_r - a_c)
            bc = lax.broadcasted_iota(jnp.int32, (RM, R), 0) // 16
            brw = lax.broadcasted_iota(jnp.int32, (RM, R), 1) // 128
            adj = (absd > 0.5) & (absd < rr + 0.5) & (bc == brw)
            adjf = adj.astype(jnp.float32)
            cnt = jnp.sum(adjf, axis=1, keepdims=True)
            M_scr[...] = adjf / jnp.maximum(cnt, 1.0)
            ha = rows_ref[R:R + RM, :]
            logits = jnp.dot(ha, Wr_ref[...],
                             preferred_element_type=jnp.float32) + br_ref[...]
            mx = jnp.max(logits, axis=1, keepdims=True)
            e = jnp.exp(logits - mx)
            w_scr[...] = e / jnp.sum(e, axis=1, keepdims=True)
            acc_scr[...] = jnp.zeros_like(acc_scr)
            wb_scr[...] = jnp.zeros_like(wb_scr)
            for z in range(NZ):
                pltpu.make_async_copy(
                    zblk, out_ref.at[z // (NZ // B)].at[
                        pl.ds((z % (NZ // B)) * ZB, ZB)], zsem.at[z]).start()

        hts = rows_ref[0:R, :]
        h1 = jnp.dot(hts, W1_ref[0],
                     preferred_element_type=jnp.float32) + b1_ref[0]
        H = h1 * 0.5 * (1.0 + lax.erf(h1 * 0.7071067811865476))
        P = jnp.dot(M_scr[...], H, preferred_element_type=jnp.float32)
        lane = lax.broadcasted_iota(jnp.int32, (RM, 128), 1)
        wcol = jnp.sum(jnp.where(lane == k, w_scr[...], 0.0),
                       axis=1, keepdims=True)
        acc_scr[...] += jnp.dot(P * wcol, W2_ref[0],
                                preferred_element_type=jnp.float32)
        wb_scr[...] += wcol * b2_ref[0]

        @pl.when(k == K - 1)
        def _fin():
            s_col = (jnp.sum(M_scr[...], axis=1, keepdims=True) > 0.5)
            eo = acc_scr[...] + s_col.astype(jnp.float32) * wb_scr[...]
            mu = jnp.mean(eo, axis=1, keepdims=True)
            var = jnp.mean((eo - mu) ** 2, axis=1, keepdims=True)
            val_scr[...] = (eo - mu) * lax.rsqrt(var + 1e-5)
            for z in range(NZ):
                pltpu.make_async_copy(
                    zblk, out_ref.at[z // (NZ // B)].at[
                        pl.ds((z % (NZ // B)) * ZB, ZB)], zsem.at[z]).wait()
            for j in range(RM):
                @pl.when(win_ref[j] != 0)
                def _row(j=j):
                    pltpu.make_async_copy(
                        val_scr.at[pl.ds(j, 1)],
                        out_ref.at[tb_ref[j]].at[pl.ds(tr_ref[j], 1)],
                        rsem.at[j]).start()
            for j in range(RM):
                @pl.when(win_ref[j] != 0)
                def _roww(j=j):
                    pltpu.make_async_copy(
                        val_scr.at[pl.ds(j, 1)],
                        out_ref.at[tb_ref[j]].at[pl.ds(tr_ref[j], 1)],
                        rsem.at[j]).wait()

    return pl.pallas_call(
        body,
        grid=(K,),
        in_specs=[
            pl.BlockSpec((R + RM, D), lambda k: (0, 0)),
            pl.BlockSpec((1, D, D4), lambda k: (k, 0, 0)),
            pl.BlockSpec((1, 1, D4), lambda k: (k, 0, 0)),
            pl.BlockSpec((1, D4, D), lambda k: (k, 0, 0)),
            pl.BlockSpec((1, 1, D), lambda k: (k, 0, 0)),
            pl.BlockSpec((D, 128), lambda k: (0, 0)),
            pl.BlockSpec((1, 128), lambda k: (0, 0)),
            pl.BlockSpec((1, R), lambda k: (0, 0)),
            pl.BlockSpec((RM, 1), lambda k: (0, 0)),
            pl.BlockSpec((1, 1), lambda k: (0, 0)),
            pl.BlockSpec(memory_space=pltpu.SMEM),
            pl.BlockSpec(memory_space=pltpu.SMEM),
            pl.BlockSpec(memory_space=pltpu.SMEM),
        ],
        out_specs=pl.BlockSpec(memory_space=pl.ANY),
        out_shape=jax.ShapeDtypeStruct((B, S, D), jnp.float32),
        scratch_shapes=[
            pltpu.VMEM((RM, R), jnp.float32),
            pltpu.VMEM((RM, 128), jnp.float32),
            pltpu.VMEM((RM, D), jnp.float32),
            pltpu.VMEM((RM, D), jnp.float32),
            pltpu.VMEM((RM, D), jnp.float32),
            pltpu.VMEM((ZB, D), jnp.float32),
            pltpu.SemaphoreType.DMA((NZ,)),
            pltpu.SemaphoreType.DMA((RM,)),
        ],
        compiler_params=pltpu.CompilerParams(
            dimension_semantics=("arbitrary",)),
    )(rows, W1, b1.reshape(K, 1, D4), W2, b2.reshape(K, 1, D),
      Wr_pad, br_pad, u_row, a_col, rrf, win32, tb32, tr32)


def _routed_mixture(rows, mask_indices, unmasked_indices, range_r, Wr, br,
                    W1, b1, W2, b2, B, S):
    K, D, _ = W1.shape
    uf = unmasked_indices.astype(jnp.float32).reshape(-1)
    af = mask_indices.astype(jnp.float32).reshape(-1)
    u_row = uf.reshape(1, -1)
    a_col = af.reshape(-1, 1)
    rrf = jnp.asarray(range_r, jnp.float32).reshape(1, 1)
    Wr_pad = jnp.concatenate([Wr, jnp.zeros((D, 128 - K), Wr.dtype)], axis=1)
    br_pad = jnp.concatenate(
        [br, jnp.full((128 - K,), -1e9, br.dtype)]).reshape(1, 128)
    a_i = mask_indices.astype(jnp.int32)
    u_i = unmasked_indices.astype(jnp.int32)
    ad = jnp.abs(a_i[:, :, None] - u_i[:, None, :])
    s = ((ad >= 1) & (ad <= range_r)).sum(-1) > 0
    jj = jnp.arange(a_i.shape[1], dtype=jnp.int32)
    eqm = a_i[:, :, None] == a_i[:, None, :]
    exists = (eqm & (jj[None, :] > jj[:, None])[None] & s[:, None, :]).any(-1)
    win32 = (s & ~exists).reshape(-1).astype(jnp.int32)
    tb32 = jnp.repeat(jnp.arange(B, dtype=jnp.int32), a_i.shape[1])
    tr32 = a_i.reshape(-1)
    return _compute_tc(rows, W1, b1, W2, b2, Wr_pad, br_pad, u_row, a_col,
                       rrf, win32, tb32, tr32, B, S)


def kernel(h_L, mask_indices, unmasked_indices, range_r, Wr, br, W1, b1, W2,
           b2):
    B, S, D = h_L.shape
    offs = (jnp.arange(B, dtype=jnp.int32) * S)[:, None]
    ug = (unmasked_indices.astype(jnp.int32) + offs).reshape(32, 8)
    ag = (mask_indices.astype(jnp.int32) + offs).reshape(32, 1)
    gidx = jnp.concatenate([ug, jnp.broadcast_to(ag, (32, 8))], axis=1)
    rows = _gather_rows_sc(h_L.reshape(B * S, D), gidx, 288)
    return _routed_mixture(rows, mask_indices, unmasked_indices, range_r, Wr,
                           br, W1, b1, W2, b2, B, S)
